# baseline (speedup 1.0000x reference)
import functools

import jax
import jax.numpy as jnp
from jax import lax
from jax.experimental import pallas as pl
from jax.experimental.pallas import tpu as pltpu
from jax.experimental.pallas import tpu_sc as plsc

VOCAB = 100000
EMBED_DIM = 64
BATCH = 4096

_info = plsc.get_sparse_core_info()
_NC = _info.num_cores
_NS = _info.num_subcores
_L = _info.num_lanes
_NW = _NC * _NS
_BPW = BATCH // _NW
_GROUPS = _BPW // _L
_CHUNKS = EMBED_DIM // _L

_mesh = plsc.VectorSubcoreMesh(core_axis_name="c", subcore_axis_name="s")


@functools.partial(
    pl.kernel,
    mesh=_mesh,
    out_type=jax.ShapeDtypeStruct((_NW * _L,), jnp.float32),
    compiler_params=pltpu.CompilerParams(use_tc_tiling_on_sc=True,
                                         needs_layout_passes=False),
    scratch_types=[
        pltpu.VMEM((_BPW,), jnp.int32),
        pltpu.VMEM((_BPW,), jnp.int32),
        pltpu.VMEM((_BPW, EMBED_DIM), jnp.float32),
        pltpu.VMEM((_BPW, EMBED_DIM), jnp.float32),
        pltpu.VMEM((_L,), jnp.float32),
        pltpu.SemaphoreType.DMA,
        pltpu.SemaphoreType.DMA,
    ],
)
def _sc_partial_sumsq(table_hbm, head_hbm, tail_hbm, out_hbm,
                      hidx_v, tidx_v, hrow_v, trow_v, acc_v, sem_h, sem_t):
    wid = lax.axis_index("s") * _NC + lax.axis_index("c")
    base = wid * _BPW
    pltpu.sync_copy(head_hbm.at[pl.ds(base, _BPW)], hidx_v)
    pltpu.sync_copy(tail_hbm.at[pl.ds(base, _BPW)], tidx_v)

    def issue(g, carry):
        hv = hidx_v[pl.ds(g * _L, _L)]
        tv = tidx_v[pl.ds(g * _L, _L)]
        for l in range(_L):
            r = g * _L + l
            pltpu.async_copy(table_hbm.at[pl.ds(hv[l], 1)],
                             hrow_v.at[pl.ds(r, 1)], sem_h)
            pltpu.async_copy(table_hbm.at[pl.ds(tv[l], 1)],
                             trow_v.at[pl.ds(r, 1)], sem_t)
        return carry

    lax.fori_loop(0, _GROUPS, issue, 0)

    pltpu.make_async_copy(table_hbm.at[pl.ds(0, _BPW)], hrow_v, sem_h).wait()
    pltpu.make_async_copy(table_hbm.at[pl.ds(0, _BPW)], trow_v, sem_t).wait()

    def body(r, accs):
        new = []
        for c in range(_CHUNKS):
            h = hrow_v[r, pl.ds(c * _L, _L)]
            t = trow_v[r, pl.ds(c * _L, _L)]
            d = h - t
            new.append(accs[c] + d * d)
        return tuple(new)

    zero = jnp.zeros((_L,), jnp.float32)
    accs = lax.fori_loop(0, _BPW, body, (zero,) * _CHUNKS)
    total = accs[0]
    for c in range(1, _CHUNKS):
        total = total + accs[c]
    acc_v[...] = total
    pltpu.sync_copy(acc_v, out_hbm.at[pl.ds(wid * _L, _L)])


def kernel(table, head, tail):
    partials = _sc_partial_sumsq(
        table, head.astype(jnp.int32), tail.astype(jnp.int32))
    steer = jnp.take(table, head[:8], axis=0)
    return jnp.sqrt(jnp.sum(partials) + 0.0 * jnp.sum(steer))


# speedup vs baseline: 1.2438x; 1.2438x over previous
import functools

import jax
import jax.numpy as jnp
from jax import lax
from jax.experimental import pallas as pl
from jax.experimental.pallas import tpu as pltpu
from jax.experimental.pallas import tpu_sc as plsc

VOCAB = 100000
EMBED_DIM = 64
BATCH = 4096

_info = plsc.get_sparse_core_info()
_NC = _info.num_cores
_NS = _info.num_subcores
_L = _info.num_lanes
_NW = _NC * _NS
_BPW = BATCH // _NW
_GROUPS = _BPW // _L
_CHUNKS = EMBED_DIM // _L

_mesh = plsc.VectorSubcoreMesh(core_axis_name="c", subcore_axis_name="s")


@functools.partial(
    pl.kernel,
    mesh=_mesh,
    out_type=jax.ShapeDtypeStruct((_NW * _L,), jnp.float32),
    compiler_params=pltpu.CompilerParams(use_tc_tiling_on_sc=True,
                                         needs_layout_passes=False),
    scratch_types=[
        pltpu.VMEM((_BPW,), jnp.int32),
        pltpu.VMEM((_BPW,), jnp.int32),
        pltpu.VMEM((_BPW, EMBED_DIM), jnp.float32),
        pltpu.VMEM((_BPW, EMBED_DIM), jnp.float32),
        pltpu.VMEM((_L,), jnp.float32),
        pltpu.SemaphoreType.DMA,
        pltpu.SemaphoreType.DMA,
        pltpu.SemaphoreType.DMA,
        pltpu.SemaphoreType.DMA,
    ],
)
def _sc_partial_sumsq(table_hbm, head_hbm, tail_hbm, out_hbm,
                      hidx_v, tidx_v, hrow_v, trow_v, acc_v,
                      sem_h, sem_t, sem_h2, sem_t2):
    wid = lax.axis_index("s") * _NC + lax.axis_index("c")
    base = wid * _BPW
    ci_h = pltpu.async_copy(head_hbm.at[pl.ds(base, _BPW)], hidx_v, sem_h)
    ci_t = pltpu.async_copy(tail_hbm.at[pl.ds(base, _BPW)], tidx_v, sem_t)
    ci_h.wait()
    ci_t.wait()

    half = _GROUPS // 2

    def issue(g, carry):
        hv = hidx_v[pl.ds(g * _L, _L)]
        tv = tidx_v[pl.ds(g * _L, _L)]
        for l in range(_L):
            r = g * _L + l
            pltpu.async_copy(table_hbm.at[pl.ds(hv[l], 1)],
                             hrow_v.at[pl.ds(r, 1)], sem_h)
            pltpu.async_copy(table_hbm.at[pl.ds(tv[l], 1)],
                             trow_v.at[pl.ds(r, 1)], sem_t)
        return carry

    lax.fori_loop(0, half, issue, 0)

    def issue2(g, carry):
        hv = hidx_v[pl.ds(g * _L, _L)]
        tv = tidx_v[pl.ds(g * _L, _L)]
        for l in range(_L):
            r = g * _L + l
            pltpu.async_copy(table_hbm.at[pl.ds(hv[l], 1)],
                             hrow_v.at[pl.ds(r, 1)], sem_h2)
            pltpu.async_copy(table_hbm.at[pl.ds(tv[l], 1)],
                             trow_v.at[pl.ds(r, 1)], sem_t2)
        return carry

    lax.fori_loop(half, _GROUPS, issue2, 0)

    _HB = half * _L

    def body(r2, accs):
        new = list(accs)
        for u in range(2):
            r = r2 * 2 + u
            for c in range(_CHUNKS):
                h = hrow_v[r, pl.ds(c * _L, _L)]
                t = trow_v[r, pl.ds(c * _L, _L)]
                d = h - t
                new[c] = new[c] + d * d
        return tuple(new)

    zero = jnp.zeros((_L,), jnp.float32)

    pltpu.make_async_copy(table_hbm.at[pl.ds(0, _HB)],
                          hrow_v.at[pl.ds(0, _HB)], sem_h).wait()
    pltpu.make_async_copy(table_hbm.at[pl.ds(0, _HB)],
                          trow_v.at[pl.ds(0, _HB)], sem_t).wait()
    accs = lax.fori_loop(0, _HB // 2, body, (zero,) * _CHUNKS)

    pltpu.make_async_copy(table_hbm.at[pl.ds(0, _BPW - _HB)],
                          hrow_v.at[pl.ds(_HB, _BPW - _HB)], sem_h2).wait()
    pltpu.make_async_copy(table_hbm.at[pl.ds(0, _BPW - _HB)],
                          trow_v.at[pl.ds(_HB, _BPW - _HB)], sem_t2).wait()
    accs = lax.fori_loop(_HB // 2, _BPW // 2, body, accs)

    total = accs[0]
    for c in range(1, _CHUNKS):
        total = total + accs[c]
    acc_v[...] = total
    pltpu.sync_copy(acc_v, out_hbm.at[pl.ds(wid * _L, _L)])


def kernel(table, head, tail):
    partials = _sc_partial_sumsq(
        table, head.astype(jnp.int32), tail.astype(jnp.int32))
    return jnp.sqrt(jnp.sum(partials))


# speedup vs baseline: 1.2445x; 1.0005x over previous
import functools

import jax
import jax.numpy as jnp
from jax import lax
from jax.experimental import pallas as pl
from jax.experimental.pallas import tpu as pltpu
from jax.experimental.pallas import tpu_sc as plsc

VOCAB = 100000
EMBED_DIM = 64
BATCH = 4096

_info = plsc.get_sparse_core_info()
_NC = _info.num_cores
_NS = _info.num_subcores
_L = _info.num_lanes
_NW = _NC * _NS
_BPW = BATCH // _NW
_GROUPS = _BPW // _L
_CHUNKS = EMBED_DIM // _L

_mesh = plsc.VectorSubcoreMesh(core_axis_name="c", subcore_axis_name="s")


@functools.partial(
    pl.kernel,
    mesh=_mesh,
    out_type=jax.ShapeDtypeStruct((_NW * _L,), jnp.float32),
    compiler_params=pltpu.CompilerParams(use_tc_tiling_on_sc=True),
    scratch_types=[
        pltpu.VMEM((_BPW,), jnp.int32),
        pltpu.VMEM((_BPW,), jnp.int32),
        pltpu.VMEM((_BPW, EMBED_DIM), jnp.float32),
        pltpu.VMEM((_BPW, EMBED_DIM), jnp.float32),
        pltpu.VMEM((_L,), jnp.float32),
        pltpu.SemaphoreType.DMA,
        pltpu.SemaphoreType.DMA,
        pltpu.SemaphoreType.DMA,
        pltpu.SemaphoreType.DMA,
    ],
)
def _sc_partial_sumsq(table_hbm, head_hbm, tail_hbm, out_hbm,
                      hidx_v, tidx_v, hrow_v, trow_v, acc_v,
                      sem_h, sem_t, sem_h2, sem_t2):
    wid = lax.axis_index("s") * _NC + lax.axis_index("c")
    base = wid * _BPW
    ci_h = pltpu.async_copy(head_hbm.at[pl.ds(base, _BPW)], hidx_v, sem_h)
    ci_t = pltpu.async_copy(tail_hbm.at[pl.ds(base, _BPW)], tidx_v, sem_t)
    ci_h.wait()
    ci_t.wait()

    half = _GROUPS // 2

    def issue(g, carry):
        hv = hidx_v[pl.ds(g * _L, _L)]
        tv = tidx_v[pl.ds(g * _L, _L)]
        for l in range(_L):
            r = g * _L + l
            pltpu.async_copy(table_hbm.at[pl.ds(hv[l], 1)],
                             hrow_v.at[pl.ds(r, 1)], sem_h)
            pltpu.async_copy(table_hbm.at[pl.ds(tv[l], 1)],
                             trow_v.at[pl.ds(r, 1)], sem_t)
        return carry

    lax.fori_loop(0, half, issue, 0)

    def issue2(g, carry):
        hv = hidx_v[pl.ds(g * _L, _L)]
        tv = tidx_v[pl.ds(g * _L, _L)]
        for l in range(_L):
            r = g * _L + l
            pltpu.async_copy(table_hbm.at[pl.ds(hv[l], 1)],
                             hrow_v.at[pl.ds(r, 1)], sem_h2)
            pltpu.async_copy(table_hbm.at[pl.ds(tv[l], 1)],
                             trow_v.at[pl.ds(r, 1)], sem_t2)
        return carry

    lax.fori_loop(half, _GROUPS, issue2, 0)

    _HB = half * _L

    def body(r2, accs):
        new = list(accs)
        for u in range(2):
            r = r2 * 2 + u
            for c in range(_CHUNKS):
                h = hrow_v[r, pl.ds(c * _L, _L)]
                t = trow_v[r, pl.ds(c * _L, _L)]
                d = h - t
                new[c] = new[c] + d * d
        return tuple(new)

    zero = jnp.zeros((_L,), jnp.float32)

    pltpu.make_async_copy(table_hbm.at[pl.ds(0, _HB)],
                          hrow_v.at[pl.ds(0, _HB)], sem_h).wait()
    pltpu.make_async_copy(table_hbm.at[pl.ds(0, _HB)],
                          trow_v.at[pl.ds(0, _HB)], sem_t).wait()
    accs = lax.fori_loop(0, _HB // 2, body, (zero,) * _CHUNKS)

    pltpu.make_async_copy(table_hbm.at[pl.ds(0, _BPW - _HB)],
                          hrow_v.at[pl.ds(_HB, _BPW - _HB)], sem_h2).wait()
    pltpu.make_async_copy(table_hbm.at[pl.ds(0, _BPW - _HB)],
                          trow_v.at[pl.ds(_HB, _BPW - _HB)], sem_t2).wait()
    accs = lax.fori_loop(_HB // 2, _BPW // 2, body, accs)

    total = accs[0]
    for c in range(1, _CHUNKS):
        total = total + accs[c]
    acc_v[...] = total
    pltpu.sync_copy(acc_v, out_hbm.at[pl.ds(wid * _L, _L)])


def kernel(table, head, tail):
    partials = _sc_partial_sumsq(
        table, head.astype(jnp.int32), tail.astype(jnp.int32))
    return jnp.sqrt(jnp.sum(partials))
